# baseline (speedup 1.0000x reference)
import functools

import jax
import jax.numpy as jnp
from jax import lax
from jax.experimental import pallas as pl

B = 4
C = 256
A = 9
AP = 16
T = 512
HWS = (16384, 4096, 1024)
NT0, NT1, NT2 = 32, 8, 2
TOT = 21504
NTOT = NT0 + NT1 + NT2
NANCH = TOT * A
K = 1000


def _conv_body(fm0_ref, fm1_ref, fm2_ref, wpre_ref, bpre_ref, wpt_ref,
               bproj_ref, lg_ref, ft_ref):
    t = pl.program_id(1)

    def compute(x):
        h = lax.dot_general(wpre_ref[...], x, (((1,), (0,)), ((), ())),
                            preferred_element_type=jnp.float32)
        h = jnp.maximum(h + bpre_ref[...], 0.0)
        lg = lax.dot_general(h, wpt_ref[...], (((0,), (0,)), ((), ())),
                             preferred_element_type=jnp.float32)
        lg_ref[0] = lg + bproj_ref[...]
        ft_ref[0] = x.T

    @pl.when(t < NT0)
    def _():
        compute(fm0_ref[0])

    @pl.when(jnp.logical_and(t >= NT0, t < NT0 + NT1))
    def _():
        compute(fm1_ref[0])

    @pl.when(t >= NT0 + NT1)
    def _():
        compute(fm2_ref[0])


def _conv_stage(fm0, fm1, fm2, W_pre, b_pre, W_proj, b_proj):
    W_projT = jnp.zeros((C, AP), jnp.float32).at[:, :A].set(W_proj.T)
    b_proj_pad = jnp.full((1, AP), -jnp.inf, jnp.float32).at[0, :A].set(b_proj)
    b_pre2d = b_pre.reshape(C, 1)

    grid = (B, NTOT)
    in_specs = [
        pl.BlockSpec((1, C, T), lambda b, t: (b, 0, jnp.minimum(t, NT0 - 1))),
        pl.BlockSpec((1, C, T),
                     lambda b, t: (b, 0, jnp.clip(t - NT0, 0, NT1 - 1))),
        pl.BlockSpec((1, C, T),
                     lambda b, t: (b, 0, jnp.clip(t - NT0 - NT1, 0, NT2 - 1))),
        pl.BlockSpec((C, C), lambda b, t: (0, 0)),
        pl.BlockSpec((C, 1), lambda b, t: (0, 0)),
        pl.BlockSpec((C, AP), lambda b, t: (0, 0)),
        pl.BlockSpec((1, AP), lambda b, t: (0, 0)),
    ]
    out_specs = [
        pl.BlockSpec((1, T, AP), lambda b, t: (b, t, 0)),
        pl.BlockSpec((1, T, C), lambda b, t: (b, t, 0)),
    ]
    out_shape = [
        jax.ShapeDtypeStruct((B, TOT, AP), jnp.float32),
        jax.ShapeDtypeStruct((B, TOT, C), jnp.float32),
    ]
    lg, ft = pl.pallas_call(
        _conv_body, grid=grid, in_specs=in_specs, out_specs=out_specs,
        out_shape=out_shape,
    )(fm0, fm1, fm2, W_pre, b_pre2d, W_projT, b_proj_pad)
    return lg, ft


def kernel(feat_map0, feat_map1, feat_map2, W_pre, b_pre, W_proj, b_proj):
    fm0 = feat_map0.reshape(B, C, HWS[0])
    fm1 = feat_map1.reshape(B, C, HWS[1])
    fm2 = feat_map2.reshape(B, C, HWS[2])
    lg, ft = _conv_stage(fm0, fm1, fm2, W_pre, b_pre, W_proj, b_proj)
    sel_logits = lg[..., :A].reshape(B, NANCH)
    sel_probs = jax.nn.sigmoid(sel_logits)
    _, rel_ids = jax.lax.top_k(sel_probs, K)
    sel_ids = (rel_ids
               + NANCH * jnp.arange(B, dtype=rel_ids.dtype)[:, None]).reshape(-1)
    feat_ids = sel_ids // A
    sel_feats = jnp.take(ft.reshape(B * TOT, C), feat_ids, axis=0)
    return sel_logits, sel_ids, sel_feats


# speedup vs baseline: 2.2618x; 2.2618x over previous
import functools

import jax
import jax.numpy as jnp
from jax import lax
from jax.experimental import pallas as pl
from jax.experimental.pallas import tpu as pltpu
from jax.experimental.pallas import tpu_sc as plsc

B = 4
C = 256
A = 9
AP = 16
T = 512
HWS = (16384, 4096, 1024)
NT0, NT1, NT2 = 32, 8, 2
TOT = 21504
NTOT = NT0 + NT1 + NT2
NANCH = TOT * A
K = 1000


def _conv_body(fm0_ref, fm1_ref, fm2_ref, wpre_ref, bpre_ref, wpt_ref,
               bproj_ref, lg_ref, ft_ref):
    t = pl.program_id(1)

    def compute(x):
        h = lax.dot_general(wpre_ref[...], x, (((1,), (0,)), ((), ())),
                            preferred_element_type=jnp.float32)
        h = jnp.maximum(h + bpre_ref[...], 0.0)
        lg = lax.dot_general(h, wpt_ref[...], (((0,), (0,)), ((), ())),
                             preferred_element_type=jnp.float32)
        lg_ref[0] = lg + bproj_ref[...]
        ft_ref[0] = x.T

    @pl.when(t < NT0)
    def _():
        compute(fm0_ref[0])

    @pl.when(jnp.logical_and(t >= NT0, t < NT0 + NT1))
    def _():
        compute(fm1_ref[0])

    @pl.when(t >= NT0 + NT1)
    def _():
        compute(fm2_ref[0])


def _conv_stage(fm0, fm1, fm2, W_pre, b_pre, W_proj, b_proj):
    W_projT = jnp.zeros((C, AP), jnp.float32).at[:, :A].set(W_proj.T)
    b_proj_pad = jnp.full((1, AP), -jnp.inf, jnp.float32).at[0, :A].set(b_proj)
    b_pre2d = b_pre.reshape(C, 1)

    grid = (B, NTOT)
    in_specs = [
        pl.BlockSpec((1, C, T), lambda b, t: (b, 0, jnp.minimum(t, NT0 - 1))),
        pl.BlockSpec((1, C, T),
                     lambda b, t: (b, 0, jnp.clip(t - NT0, 0, NT1 - 1))),
        pl.BlockSpec((1, C, T),
                     lambda b, t: (b, 0, jnp.clip(t - NT0 - NT1, 0, NT2 - 1))),
        pl.BlockSpec((C, C), lambda b, t: (0, 0)),
        pl.BlockSpec((C, 1), lambda b, t: (0, 0)),
        pl.BlockSpec((C, AP), lambda b, t: (0, 0)),
        pl.BlockSpec((1, AP), lambda b, t: (0, 0)),
    ]
    out_specs = [
        pl.BlockSpec((1, T, AP), lambda b, t: (b, t, 0)),
        pl.BlockSpec((1, T, C), lambda b, t: (b, t, 0)),
    ]
    out_shape = [
        jax.ShapeDtypeStruct((B, TOT, AP), jnp.float32),
        jax.ShapeDtypeStruct((B, TOT, C), jnp.float32),
    ]
    lg, ft = pl.pallas_call(
        _conv_body, grid=grid, in_specs=in_specs, out_specs=out_specs,
        out_shape=out_shape,
    )(fm0, fm1, fm2, W_pre, b_pre2d, W_projT, b_proj_pad)
    return lg, ft


_ROWS = NANCH // 128
_RCH = _ROWS // 8
_HI0 = 0x3F800001


def _thr_body(p_ref, t_ref):
    def count_ge(mid):
        def cbody(j, acc):
            ch = lax.bitcast_convert_type(
                p_ref[0, pl.ds(j * 8, 8), :], jnp.int32)
            return acc + jnp.where(ch >= mid, 1, 0).astype(jnp.int32)
        acc = lax.fori_loop(0, _RCH, cbody,
                            jnp.zeros((8, 128), jnp.int32))
        return jnp.sum(acc)

    def sbody(_, lohi):
        lo, hi = lohi
        mid = (lo + hi) // 2
        c = count_ge(mid)
        big = c >= K
        return jnp.where(big, mid, lo), jnp.where(big, hi, mid)

    lo, _ = lax.fori_loop(0, 31, sbody,
                          (jnp.int32(0), jnp.int32(_HI0)))
    t_ref[0] = jnp.full((1, 16), lo, jnp.int32)


def _thr_stage(probs):
    p3 = probs.reshape(B, _ROWS, 128)
    thr = pl.pallas_call(
        _thr_body,
        grid=(B,),
        in_specs=[pl.BlockSpec((1, _ROWS, 128), lambda b: (b, 0, 0))],
        out_specs=pl.BlockSpec((1, 1, 16), lambda b: (b, 0, 0)),
        out_shape=jax.ShapeDtypeStruct((B, 1, 16), jnp.int32),
    )(p3)
    return thr.reshape(B, 16)


CAP = 2048
CHUNK = NANCH // 8
NV = CHUNK // 16
OUTK = 1024
ROWPAD = 1040


def _sc_stage(probs, thr, feats):
    mesh = plsc.VectorSubcoreMesh(core_axis_name="c", subcore_axis_name="s")

    @functools.partial(
        pl.kernel, mesh=mesh,
        compiler_params=pltpu.CompilerParams(needs_layout_passes=False),
        out_type=[
            jax.ShapeDtypeStruct((B * OUTK,), jnp.int32),
            jax.ShapeDtypeStruct((B * OUTK + 16, C), jnp.float32),
        ],
        scratch_types=[
            pltpu.VMEM((CHUNK,), jnp.int32),
            pltpu.VMEM((CAP + 16,), jnp.int32),
            pltpu.VMEM((CAP + 16,), jnp.int32),
            pltpu.VMEM((8, CAP), jnp.int32),
            pltpu.VMEM((8, CAP), jnp.int32),
            pltpu.VMEM((8, 16), jnp.int32),
            pltpu.VMEM((16, C), jnp.float32),
            pltpu.VMEM((16,), jnp.int32),
            pltpu.VMEM((16,), jnp.int32),
            pltpu.VMEM((OUTK,), jnp.int32),
            pltpu.VMEM_SHARED((16, CAP), jnp.int32),
            pltpu.VMEM_SHARED((16, CAP), jnp.int32),
            pltpu.VMEM_SHARED((16, 16), jnp.int32),
            pltpu.VMEM_SHARED((2 * ROWPAD,), jnp.int32),
            pltpu.SemaphoreType.DMA,
        ],
    )
    def sc_kernel(probs_hbm, thr_hbm, feats_hbm, ids_hbm, feats_out_hbm,
                  keys_v, candk_v, candi_v, pubk_v, pubi_v, cnt8_v,
                  rows_v, stage_v, thr_v, idsout_v, spm_k, spm_i, spm_c,
                  spm_ids, sem):
        cid = lax.axis_index("c")
        sid = lax.axis_index("s")
        rl = sid // 8
        row = cid * 2 + rl
        slot = sid % 8
        sbase = rl * 8

        iota = lax.iota(jnp.int32, 16)

        pltpu.sync_copy(probs_hbm.at[row, pl.ds(slot * CHUNK, CHUNK)], keys_v)
        pltpu.sync_copy(thr_hbm.at[row], thr_v)
        tval = thr_v[...][0]
        tvec = jnp.full((16,), tval, jnp.int32)

        neg1 = jnp.full((16,), -1, jnp.int32)

        def fbody(i, _):
            candk_v[pl.ds(i * 16, 16)] = neg1
            candi_v[pl.ds(i * 16, 16)] = neg1
            return 0
        lax.fori_loop(0, CAP // 16 + 1, fbody, 0)

        base_ids = jnp.full((16,), slot * CHUNK, jnp.int32) + iota

        def cbody(i, off):
            ki = keys_v[pl.ds(i * 16, 16)]
            m = ki >= tvec
            cnt = plsc.all_reduce_population_count(m)[0]
            offc = jnp.minimum(off, CAP - 16)
            plsc.store_compressed(candk_v.at[pl.ds(offc, 16)], ki, mask=m)
            ids = base_ids + jnp.full((16,), i * 16, jnp.int32)
            plsc.store_compressed(candi_v.at[pl.ds(offc, 16)], ids, mask=m)
            return off + cnt

        n_own = lax.fori_loop(0, NV, cbody, jnp.int32(0))
        n_own = jnp.minimum(n_own, CAP)

        stage_v[...] = jnp.where(iota == 0, n_own, 0)
        pltpu.sync_copy(stage_v, spm_c.at[sid])
        pltpu.sync_copy(candk_v.at[pl.ds(0, CAP)], spm_k.at[sid])
        pltpu.sync_copy(candi_v.at[pl.ds(0, CAP)], spm_i.at[sid])
        plsc.subcore_barrier()
        for j in range(8):
            pltpu.sync_copy(spm_k.at[sbase + j], pubk_v.at[j])
            pltpu.sync_copy(spm_i.at[sbase + j], pubi_v.at[j])
            pltpu.sync_copy(spm_c.at[sbase + j], cnt8_v.at[j])

        trips = []
        for j in range(8):
            nj = cnt8_v[j][0]
            trips.append((nj + 15) // 16)

        kvec = jnp.full((16,), K, jnp.int32)
        ninth = jnp.full((16,), jnp.float32(1.0 / 9.0))
        half = jnp.full((16,), jnp.float32(0.5))

        def qbody(q, rkvec):
            myk = jnp.full((16,), candk_v[pl.ds(q, 16)][0], jnp.int32)
            myi = jnp.full((16,), candi_v[pl.ds(q, 16)][0], jnp.int32)
            acc = jnp.zeros((16,), jnp.int32)
            for j in range(8):
                def ibody(v, a, j=j):
                    pk = pubk_v[j, pl.ds(v * 16, 16)]
                    pi = pubi_v[j, pl.ds(v * 16, 16)]
                    better = jnp.logical_or(
                        pk > myk, jnp.logical_and(pk == myk, pi < myi))
                    return a + jnp.where(better, 1, 0).astype(jnp.int32)

                acc = lax.fori_loop(0, trips[j], ibody, acc)
            rank_l = jnp.sum(acc)
            lane = lax.rem(q, 16)
            rkvec = jnp.where(iota == jnp.full((16,), lane, jnp.int32),
                              jnp.full((16,), rank_l, jnp.int32), rkvec)

            @pl.when(lane == 15)
            def _():
                rk = rkvec
                ids = candi_v[pl.ds(q - 15, 16)]
                ok = rk < kvec
                slot_idx = jnp.where(
                    ok, jnp.full((16,), rl * ROWPAD, jnp.int32) + rk,
                    jnp.full((16,), rl * ROWPAD + OUTK, jnp.int32) + iota)
                stage_v[...] = jnp.full((16,), row * NANCH, jnp.int32) + ids
                pltpu.async_copy(stage_v, spm_ids.at[slot_idx], sem).wait()
                fidx = ((ids.astype(jnp.float32) + half) * ninth
                        ).astype(jnp.int32) + jnp.full((16,), row * TOT,
                                                       jnp.int32)
                pltpu.async_copy(feats_hbm.at[fidx], rows_v, sem).wait()
                out_idx = jnp.where(
                    ok, jnp.full((16,), row * OUTK, jnp.int32) + rk,
                    jnp.full((16,), B * OUTK, jnp.int32))
                pltpu.async_copy(rows_v, feats_out_hbm.at[out_idx],
                                 sem).wait()

            return rkvec

        n_pad = ((n_own + 15) // 16) * 16
        lax.fori_loop(0, n_pad, qbody, jnp.zeros((16,), jnp.int32))

        plsc.subcore_barrier()

        @pl.when(slot == 0)
        def _():
            pltpu.sync_copy(spm_ids.at[pl.ds(rl * ROWPAD, OUTK)], idsout_v)
            pltpu.sync_copy(idsout_v, ids_hbm.at[pl.ds(row * OUTK, OUTK)])

    return sc_kernel(lax.bitcast_convert_type(probs, jnp.int32), thr, feats)


def kernel(feat_map0, feat_map1, feat_map2, W_pre, b_pre, W_proj, b_proj):
    fm0 = feat_map0.reshape(B, C, HWS[0])
    fm1 = feat_map1.reshape(B, C, HWS[1])
    fm2 = feat_map2.reshape(B, C, HWS[2])
    lg, ft = _conv_stage(fm0, fm1, fm2, W_pre, b_pre, W_proj, b_proj)
    sel_logits = lg[..., :A].reshape(B, NANCH)
    probs = jax.nn.sigmoid(sel_logits)
    thr = _thr_stage(probs)
    ids_pad, feats_pad = _sc_stage(probs, thr, ft.reshape(B * TOT, C))
    sel_ids = ids_pad.reshape(B, OUTK)[:, :K].reshape(-1)
    sel_feats = feats_pad[:B * OUTK].reshape(B, OUTK, C)[:, :K].reshape(
        B * K, C)
    return sel_logits, sel_ids, sel_feats


# speedup vs baseline: 3.0211x; 1.3357x over previous
import functools

import jax
import jax.numpy as jnp
from jax import lax
from jax.experimental import pallas as pl
from jax.experimental.pallas import tpu as pltpu
from jax.experimental.pallas import tpu_sc as plsc

B = 4
C = 256
A = 9
AP = 16
T = 1024
HWS = (16384, 4096, 1024)
NT0, NT1, NT2 = 16, 4, 1
TOT = 21504
NTOT = NT0 + NT1 + NT2
NANCH = TOT * A
K = 1000


def _conv_body(fm0_ref, fm1_ref, fm2_ref, wpre_ref, bpre_ref, wpt_ref,
               bproj_ref, lg_ref, ft_ref):
    t = pl.program_id(1)

    def compute(x):
        h = lax.dot_general(wpre_ref[...], x, (((1,), (0,)), ((), ())),
                            preferred_element_type=jnp.float32)
        h = jnp.maximum(h + bpre_ref[...], 0.0)
        lg = lax.dot_general(h, wpt_ref[...], (((0,), (0,)), ((), ())),
                             preferred_element_type=jnp.float32)
        lg_ref[0] = lg + bproj_ref[...]
        ft_ref[0] = x.T

    @pl.when(t < NT0)
    def _():
        compute(fm0_ref[0])

    @pl.when(jnp.logical_and(t >= NT0, t < NT0 + NT1))
    def _():
        compute(fm1_ref[0])

    @pl.when(t >= NT0 + NT1)
    def _():
        compute(fm2_ref[0])


def _conv_stage(fm0, fm1, fm2, W_pre, b_pre, W_proj, b_proj):
    W_projT = jnp.zeros((C, AP), jnp.float32).at[:, :A].set(W_proj.T)
    b_proj_pad = jnp.full((1, AP), -jnp.inf, jnp.float32).at[0, :A].set(b_proj)
    b_pre2d = b_pre.reshape(C, 1)

    grid = (B, NTOT)
    in_specs = [
        pl.BlockSpec((1, C, T), lambda b, t: (b, 0, jnp.minimum(t, NT0 - 1))),
        pl.BlockSpec((1, C, T),
                     lambda b, t: (b, 0, jnp.clip(t - NT0, 0, NT1 - 1))),
        pl.BlockSpec((1, C, T),
                     lambda b, t: (b, 0, jnp.clip(t - NT0 - NT1, 0, NT2 - 1))),
        pl.BlockSpec((C, C), lambda b, t: (0, 0)),
        pl.BlockSpec((C, 1), lambda b, t: (0, 0)),
        pl.BlockSpec((C, AP), lambda b, t: (0, 0)),
        pl.BlockSpec((1, AP), lambda b, t: (0, 0)),
    ]
    out_specs = [
        pl.BlockSpec((1, T, AP), lambda b, t: (b, t, 0)),
        pl.BlockSpec((1, T, C), lambda b, t: (b, t, 0)),
    ]
    out_shape = [
        jax.ShapeDtypeStruct((B, TOT, AP), jnp.float32),
        jax.ShapeDtypeStruct((B, TOT, C), jnp.float32),
    ]
    lg, ft = pl.pallas_call(
        _conv_body, grid=grid, in_specs=in_specs, out_specs=out_specs,
        out_shape=out_shape,
    )(fm0, fm1, fm2, W_pre, b_pre2d, W_projT, b_proj_pad)
    return lg, ft


_ROWS = NANCH // 128
_RCH = _ROWS // 56
_HI0 = 0x3F800001


def _thr_body(p_ref, t_ref):
    def count_ge(mid):
        def cbody(j, acc):
            ch = lax.bitcast_convert_type(
                p_ref[0, pl.ds(j * 56, 56), :], jnp.int32)
            return acc + jnp.where(ch >= mid, 1, 0).astype(jnp.int32)
        acc = lax.fori_loop(0, _RCH, cbody,
                            jnp.zeros((56, 128), jnp.int32))
        return jnp.sum(acc)

    def sbody(_, lohi):
        lo, hi = lohi
        mid = (lo + hi) // 2
        c = count_ge(mid)
        big = c >= K
        return jnp.where(big, mid, lo), jnp.where(big, hi, mid)

    lo, _ = lax.fori_loop(0, 31, sbody,
                          (jnp.int32(0), jnp.int32(_HI0)))
    t_ref[0] = jnp.full((1, 16), lo, jnp.int32)


def _thr_stage(probs):
    p3 = probs.reshape(B, _ROWS, 128)
    thr = pl.pallas_call(
        _thr_body,
        grid=(B,),
        in_specs=[pl.BlockSpec((1, _ROWS, 128), lambda b: (b, 0, 0))],
        out_specs=pl.BlockSpec((1, 1, 16), lambda b: (b, 0, 0)),
        out_shape=jax.ShapeDtypeStruct((B, 1, 16), jnp.int32),
    )(p3)
    return thr.reshape(B, 16)


CAP = 2048
CHUNK = NANCH // 8
NV = CHUNK // 16
OUTK = 1024
ROWPAD = 1040


def _sc_stage(probs, thr, feats):
    mesh = plsc.VectorSubcoreMesh(core_axis_name="c", subcore_axis_name="s")

    @functools.partial(
        pl.kernel, mesh=mesh,
        compiler_params=pltpu.CompilerParams(needs_layout_passes=False),
        out_type=[
            jax.ShapeDtypeStruct((B * OUTK,), jnp.int32),
            jax.ShapeDtypeStruct((B * OUTK + 16, C), jnp.float32),
        ],
        scratch_types=[
            pltpu.VMEM((CHUNK,), jnp.int32),
            pltpu.VMEM((CAP + 16,), jnp.int32),
            pltpu.VMEM((CAP + 16,), jnp.int32),
            pltpu.VMEM((8, CAP), jnp.int32),
            pltpu.VMEM((8, CAP), jnp.int32),
            pltpu.VMEM((8, 16), jnp.int32),
            pltpu.VMEM((16, C), jnp.float32),
            pltpu.VMEM((16,), jnp.int32),
            pltpu.VMEM((16,), jnp.int32),
            pltpu.VMEM((OUTK,), jnp.int32),
            pltpu.VMEM_SHARED((16, CAP), jnp.int32),
            pltpu.VMEM_SHARED((16, CAP), jnp.int32),
            pltpu.VMEM_SHARED((16, 16), jnp.int32),
            pltpu.VMEM_SHARED((2 * ROWPAD,), jnp.int32),
            pltpu.SemaphoreType.DMA,
        ],
    )
    def sc_kernel(probs_hbm, thr_hbm, feats_hbm, ids_hbm, feats_out_hbm,
                  keys_v, candk_v, candi_v, pubk_v, pubi_v, cnt8_v,
                  rows_v, stage_v, thr_v, idsout_v, spm_k, spm_i, spm_c,
                  spm_ids, sem):
        cid = lax.axis_index("c")
        sid = lax.axis_index("s")
        rl = sid // 8
        row = cid * 2 + rl
        slot = sid % 8
        sbase = rl * 8

        iota = lax.iota(jnp.int32, 16)

        pltpu.sync_copy(probs_hbm.at[row, pl.ds(slot * CHUNK, CHUNK)], keys_v)
        pltpu.sync_copy(thr_hbm.at[row], thr_v)
        tval = thr_v[...][0]
        tvec = jnp.full((16,), tval, jnp.int32)

        neg1 = jnp.full((16,), -1, jnp.int32)

        def fbody(i, _):
            candk_v[pl.ds(i * 16, 16)] = neg1
            candi_v[pl.ds(i * 16, 16)] = neg1
            return 0
        lax.fori_loop(0, CAP // 16 + 1, fbody, 0)

        base_ids = jnp.full((16,), slot * CHUNK, jnp.int32) + iota

        def cbody(i, off):
            ki = keys_v[pl.ds(i * 16, 16)]
            m = ki >= tvec
            cnt = plsc.all_reduce_population_count(m)[0]
            offc = jnp.minimum(off, CAP - 16)
            plsc.store_compressed(candk_v.at[pl.ds(offc, 16)], ki, mask=m)
            ids = base_ids + jnp.full((16,), i * 16, jnp.int32)
            plsc.store_compressed(candi_v.at[pl.ds(offc, 16)], ids, mask=m)
            return off + cnt

        n_own = lax.fori_loop(0, NV, cbody, jnp.int32(0))
        n_own = jnp.minimum(n_own, CAP)

        stage_v[...] = jnp.where(iota == 0, n_own, 0)
        pltpu.sync_copy(stage_v, spm_c.at[sid])
        pltpu.sync_copy(candk_v.at[pl.ds(0, CAP)], spm_k.at[sid])
        pltpu.sync_copy(candi_v.at[pl.ds(0, CAP)], spm_i.at[sid])
        plsc.subcore_barrier()
        for j in range(8):
            pltpu.sync_copy(spm_k.at[sbase + j], pubk_v.at[j])
            pltpu.sync_copy(spm_i.at[sbase + j], pubi_v.at[j])
            pltpu.sync_copy(spm_c.at[sbase + j], cnt8_v.at[j])

        trips = []
        for j in range(8):
            nj = cnt8_v[j][0]
            trips.append((nj + 15) // 16)

        kvec = jnp.full((16,), K, jnp.int32)
        ninth = jnp.full((16,), jnp.float32(1.0 / 9.0))
        half = jnp.full((16,), jnp.float32(0.5))

        def qbody(q, rkvec):
            myk = jnp.full((16,), candk_v[pl.ds(q, 16)][0], jnp.int32)
            myi = jnp.full((16,), candi_v[pl.ds(q, 16)][0], jnp.int32)
            acc = jnp.zeros((16,), jnp.int32)
            for j in range(8):
                def ibody(v, a, j=j):
                    pk = pubk_v[j, pl.ds(v * 16, 16)]
                    pi = pubi_v[j, pl.ds(v * 16, 16)]
                    better = jnp.logical_or(
                        pk > myk, jnp.logical_and(pk == myk, pi < myi))
                    return a + jnp.where(better, 1, 0).astype(jnp.int32)

                acc = lax.fori_loop(0, trips[j], ibody, acc)
            rank_l = jnp.sum(acc)
            lane = lax.rem(q, 16)
            rkvec = jnp.where(iota == jnp.full((16,), lane, jnp.int32),
                              jnp.full((16,), rank_l, jnp.int32), rkvec)

            @pl.when(lane == 15)
            def _():
                rk = rkvec
                ids = candi_v[pl.ds(q - 15, 16)]
                ok = rk < kvec
                slot_idx = jnp.where(
                    ok, jnp.full((16,), rl * ROWPAD, jnp.int32) + rk,
                    jnp.full((16,), rl * ROWPAD + OUTK, jnp.int32) + iota)
                stage_v[...] = jnp.full((16,), row * NANCH, jnp.int32) + ids
                pltpu.async_copy(stage_v, spm_ids.at[slot_idx], sem).wait()
                fidx = ((ids.astype(jnp.float32) + half) * ninth
                        ).astype(jnp.int32) + jnp.full((16,), row * TOT,
                                                       jnp.int32)
                pltpu.async_copy(feats_hbm.at[fidx], rows_v, sem).wait()
                out_idx = jnp.where(
                    ok, jnp.full((16,), row * OUTK, jnp.int32) + rk,
                    jnp.full((16,), B * OUTK, jnp.int32))
                pltpu.async_copy(rows_v, feats_out_hbm.at[out_idx],
                                 sem).wait()

            return rkvec

        n_pad = ((n_own + 15) // 16) * 16
        lax.fori_loop(0, n_pad, qbody, jnp.zeros((16,), jnp.int32))

        plsc.subcore_barrier()

        @pl.when(slot == 0)
        def _():
            pltpu.sync_copy(spm_ids.at[pl.ds(rl * ROWPAD, OUTK)], idsout_v)
            pltpu.sync_copy(idsout_v, ids_hbm.at[pl.ds(row * OUTK, OUTK)])

    return sc_kernel(lax.bitcast_convert_type(probs, jnp.int32), thr, feats)


def kernel(feat_map0, feat_map1, feat_map2, W_pre, b_pre, W_proj, b_proj):
    fm0 = feat_map0.reshape(B, C, HWS[0])
    fm1 = feat_map1.reshape(B, C, HWS[1])
    fm2 = feat_map2.reshape(B, C, HWS[2])
    lg, ft = _conv_stage(fm0, fm1, fm2, W_pre, b_pre, W_proj, b_proj)
    sel_logits = lg[..., :A].reshape(B, NANCH)
    probs = jax.nn.sigmoid(sel_logits)
    thr = _thr_stage(probs)
    ids_pad, feats_pad = _sc_stage(probs, thr, ft.reshape(B * TOT, C))
    sel_ids = ids_pad.reshape(B, OUTK)[:, :K].reshape(-1)
    sel_feats = feats_pad[:B * OUTK].reshape(B, OUTK, C)[:, :K].reshape(
        B * K, C)
    return sel_logits, sel_ids, sel_feats


# speedup vs baseline: 3.0218x; 1.0002x over previous
import functools

import jax
import jax.numpy as jnp
from jax import lax
from jax.experimental import pallas as pl
from jax.experimental.pallas import tpu as pltpu
from jax.experimental.pallas import tpu_sc as plsc

B = 4
C = 256
A = 9
AP = 16
T = 1024
HWS = (16384, 4096, 1024)
NT0, NT1, NT2 = 16, 4, 1
TOT = 21504
NTOT = NT0 + NT1 + NT2
NANCH = TOT * A
K = 1000


def _conv_body(fm0_ref, fm1_ref, fm2_ref, wpre_ref, bpre_ref, wpt_ref,
               bproj_ref, lg_ref, ft_ref):
    t = pl.program_id(1)

    def compute(x):
        h = lax.dot_general(wpre_ref[...], x, (((1,), (0,)), ((), ())),
                            preferred_element_type=jnp.float32)
        h = jnp.maximum(h + bpre_ref[...], 0.0)
        lg = lax.dot_general(h, wpt_ref[...], (((0,), (0,)), ((), ())),
                             preferred_element_type=jnp.float32)
        lg_ref[0] = lg + bproj_ref[...]
        ft_ref[0] = x.T

    @pl.when(t < NT0)
    def _():
        compute(fm0_ref[0])

    @pl.when(jnp.logical_and(t >= NT0, t < NT0 + NT1))
    def _():
        compute(fm1_ref[0])

    @pl.when(t >= NT0 + NT1)
    def _():
        compute(fm2_ref[0])


def _conv_stage(fm0, fm1, fm2, W_pre, b_pre, W_proj, b_proj):
    W_projT = jnp.zeros((C, AP), jnp.float32).at[:, :A].set(W_proj.T)
    b_proj_pad = jnp.full((1, AP), -jnp.inf, jnp.float32).at[0, :A].set(b_proj)
    b_pre2d = b_pre.reshape(C, 1)

    grid = (B, NTOT)
    in_specs = [
        pl.BlockSpec((1, C, T), lambda b, t: (b, 0, jnp.minimum(t, NT0 - 1))),
        pl.BlockSpec((1, C, T),
                     lambda b, t: (b, 0, jnp.clip(t - NT0, 0, NT1 - 1))),
        pl.BlockSpec((1, C, T),
                     lambda b, t: (b, 0, jnp.clip(t - NT0 - NT1, 0, NT2 - 1))),
        pl.BlockSpec((C, C), lambda b, t: (0, 0)),
        pl.BlockSpec((C, 1), lambda b, t: (0, 0)),
        pl.BlockSpec((C, AP), lambda b, t: (0, 0)),
        pl.BlockSpec((1, AP), lambda b, t: (0, 0)),
    ]
    out_specs = [
        pl.BlockSpec((1, T, AP), lambda b, t: (b, t, 0)),
        pl.BlockSpec((1, T, C), lambda b, t: (b, t, 0)),
    ]
    out_shape = [
        jax.ShapeDtypeStruct((B, TOT, AP), jnp.float32),
        jax.ShapeDtypeStruct((B, TOT, C), jnp.float32),
    ]
    lg, ft = pl.pallas_call(
        _conv_body, grid=grid, in_specs=in_specs, out_specs=out_specs,
        out_shape=out_shape,
        compiler_params=pltpu.CompilerParams(
            dimension_semantics=("arbitrary", "arbitrary")),
    )(fm0, fm1, fm2, W_pre, b_pre2d, W_projT, b_proj_pad)
    return lg, ft


_ROWS = NANCH // 128
_RCH = _ROWS // 56
_HI0 = 0x3F800001


def _thr_body(p_ref, t_ref):
    def count_ge(mid):
        def cbody(j, acc):
            ch = lax.bitcast_convert_type(
                p_ref[0, pl.ds(j * 56, 56), :], jnp.int32)
            return acc + jnp.where(ch >= mid, 1, 0).astype(jnp.int32)
        acc = lax.fori_loop(0, _RCH, cbody,
                            jnp.zeros((56, 128), jnp.int32))
        return jnp.sum(acc)

    def sbody(_, lohi):
        lo, hi = lohi
        mid = (lo + hi) // 2
        c = count_ge(mid)
        big = c >= K
        return jnp.where(big, mid, lo), jnp.where(big, hi, mid)

    lo, _ = lax.fori_loop(0, 31, sbody,
                          (jnp.int32(0), jnp.int32(_HI0)))
    t_ref[0] = jnp.full((1, 16), lo, jnp.int32)


def _thr_stage(probs):
    p3 = probs.reshape(B, _ROWS, 128)
    thr = pl.pallas_call(
        _thr_body,
        grid=(B,),
        in_specs=[pl.BlockSpec((1, _ROWS, 128), lambda b: (b, 0, 0))],
        out_specs=pl.BlockSpec((1, 1, 16), lambda b: (b, 0, 0)),
        out_shape=jax.ShapeDtypeStruct((B, 1, 16), jnp.int32),
    )(p3)
    return thr.reshape(B, 16)


CAP = 2048
CHUNK = NANCH // 8
NV = CHUNK // 16
OUTK = 1024
ROWPAD = 1040


def _sc_stage(probs, thr, feats):
    mesh = plsc.VectorSubcoreMesh(core_axis_name="c", subcore_axis_name="s")

    @functools.partial(
        pl.kernel, mesh=mesh,
        compiler_params=pltpu.CompilerParams(needs_layout_passes=False),
        out_type=[
            jax.ShapeDtypeStruct((B * K,), jnp.int32),
            jax.ShapeDtypeStruct((B * OUTK + 16, C), jnp.float32),
        ],
        scratch_types=[
            pltpu.VMEM((CHUNK,), jnp.int32),
            pltpu.VMEM((CAP + 16,), jnp.int32),
            pltpu.VMEM((CAP + 16,), jnp.int32),
            pltpu.VMEM((8, CAP), jnp.int32),
            pltpu.VMEM((8, CAP), jnp.int32),
            pltpu.VMEM((8, 16), jnp.int32),
            pltpu.VMEM((16, C), jnp.float32),
            pltpu.VMEM((16,), jnp.int32),
            pltpu.VMEM((16,), jnp.int32),
            pltpu.VMEM((K,), jnp.int32),
            pltpu.VMEM_SHARED((16, CAP), jnp.int32),
            pltpu.VMEM_SHARED((16, CAP), jnp.int32),
            pltpu.VMEM_SHARED((16, 16), jnp.int32),
            pltpu.VMEM_SHARED((2 * ROWPAD,), jnp.int32),
            pltpu.SemaphoreType.DMA,
        ],
    )
    def sc_kernel(probs_hbm, thr_hbm, feats_hbm, ids_hbm, feats_out_hbm,
                  keys_v, candk_v, candi_v, pubk_v, pubi_v, cnt8_v,
                  rows_v, stage_v, thr_v, idsout_v, spm_k, spm_i, spm_c,
                  spm_ids, sem):
        cid = lax.axis_index("c")
        sid = lax.axis_index("s")
        rl = sid // 8
        row = cid * 2 + rl
        slot = sid % 8
        sbase = rl * 8

        iota = lax.iota(jnp.int32, 16)

        pltpu.sync_copy(probs_hbm.at[row, pl.ds(slot * CHUNK, CHUNK)], keys_v)
        pltpu.sync_copy(thr_hbm.at[row], thr_v)
        tval = thr_v[...][0]
        tvec = jnp.full((16,), tval, jnp.int32)

        neg1 = jnp.full((16,), -1, jnp.int32)

        def fbody(i, _):
            candk_v[pl.ds(i * 16, 16)] = neg1
            candi_v[pl.ds(i * 16, 16)] = neg1
            return 0
        lax.fori_loop(0, CAP // 16 + 1, fbody, 0)

        base_ids = jnp.full((16,), slot * CHUNK, jnp.int32) + iota

        def cbody(i, off):
            ki = keys_v[pl.ds(i * 16, 16)]
            m = ki >= tvec
            cnt = plsc.all_reduce_population_count(m)[0]
            offc = jnp.minimum(off, CAP - 16)
            plsc.store_compressed(candk_v.at[pl.ds(offc, 16)], ki, mask=m)
            ids = base_ids + jnp.full((16,), i * 16, jnp.int32)
            plsc.store_compressed(candi_v.at[pl.ds(offc, 16)], ids, mask=m)
            return off + cnt

        n_own = lax.fori_loop(0, NV, cbody, jnp.int32(0))
        n_own = jnp.minimum(n_own, CAP)

        stage_v[...] = jnp.where(iota == 0, n_own, 0)
        pltpu.sync_copy(stage_v, spm_c.at[sid])
        pltpu.sync_copy(candk_v.at[pl.ds(0, CAP)], spm_k.at[sid])
        pltpu.sync_copy(candi_v.at[pl.ds(0, CAP)], spm_i.at[sid])
        plsc.subcore_barrier()
        for j in range(8):
            pltpu.sync_copy(spm_k.at[sbase + j], pubk_v.at[j])
            pltpu.sync_copy(spm_i.at[sbase + j], pubi_v.at[j])
            pltpu.sync_copy(spm_c.at[sbase + j], cnt8_v.at[j])

        trips = []
        for j in range(8):
            nj = cnt8_v[j][0]
            trips.append((nj + 15) // 16)

        kvec = jnp.full((16,), K, jnp.int32)
        ninth = jnp.full((16,), jnp.float32(1.0 / 9.0))
        half = jnp.full((16,), jnp.float32(0.5))

        def qbody(q, rkvec):
            myk = jnp.full((16,), candk_v[pl.ds(q, 16)][0], jnp.int32)
            myi = jnp.full((16,), candi_v[pl.ds(q, 16)][0], jnp.int32)
            acc = jnp.zeros((16,), jnp.int32)
            for j in range(8):
                def ibody(v, a, j=j):
                    pk = pubk_v[j, pl.ds(v * 16, 16)]
                    pi = pubi_v[j, pl.ds(v * 16, 16)]
                    better = jnp.logical_or(
                        pk > myk, jnp.logical_and(pk == myk, pi < myi))
                    return a + jnp.where(better, 1, 0).astype(jnp.int32)

                acc = lax.fori_loop(0, trips[j], ibody, acc)
            rank_l = jnp.sum(acc)
            lane = lax.rem(q, 16)
            rkvec = jnp.where(iota == jnp.full((16,), lane, jnp.int32),
                              jnp.full((16,), rank_l, jnp.int32), rkvec)

            @pl.when(lane == 15)
            def _():
                rk = rkvec
                ids = candi_v[pl.ds(q - 15, 16)]
                ok = rk < kvec
                slot_idx = jnp.where(
                    ok, jnp.full((16,), rl * ROWPAD, jnp.int32) + rk,
                    jnp.full((16,), rl * ROWPAD + OUTK, jnp.int32) + iota)
                stage_v[...] = jnp.full((16,), row * NANCH, jnp.int32) + ids
                pltpu.async_copy(stage_v, spm_ids.at[slot_idx], sem).wait()
                fidx = ((ids.astype(jnp.float32) + half) * ninth
                        ).astype(jnp.int32) + jnp.full((16,), row * TOT,
                                                       jnp.int32)
                pltpu.async_copy(feats_hbm.at[fidx], rows_v, sem).wait()
                out_idx = jnp.where(
                    ok, jnp.full((16,), row * OUTK, jnp.int32) + rk,
                    jnp.full((16,), B * OUTK, jnp.int32))
                pltpu.async_copy(rows_v, feats_out_hbm.at[out_idx],
                                 sem).wait()

            return rkvec

        n_pad = ((n_own + 15) // 16) * 16
        lax.fori_loop(0, n_pad, qbody, jnp.zeros((16,), jnp.int32))

        plsc.subcore_barrier()

        @pl.when(slot == 0)
        def _():
            pltpu.sync_copy(spm_ids.at[pl.ds(rl * ROWPAD, K)], idsout_v)
            pltpu.sync_copy(idsout_v, ids_hbm.at[pl.ds(row * K, K)])

    return sc_kernel(lax.bitcast_convert_type(probs, jnp.int32), thr, feats)


def kernel(feat_map0, feat_map1, feat_map2, W_pre, b_pre, W_proj, b_proj):
    fm0 = feat_map0.reshape(B, C, HWS[0])
    fm1 = feat_map1.reshape(B, C, HWS[1])
    fm2 = feat_map2.reshape(B, C, HWS[2])
    lg, ft = _conv_stage(fm0, fm1, fm2, W_pre, b_pre, W_proj, b_proj)
    sel_logits = lg[..., :A].reshape(B, NANCH)
    probs = jax.nn.sigmoid(sel_logits)
    thr = _thr_stage(probs)
    ids_pad, feats_pad = _sc_stage(probs, thr, ft.reshape(B * TOT, C))
    sel_ids = ids_pad
    sel_feats = feats_pad[:B * OUTK].reshape(B, OUTK, C)[:, :K].reshape(
        B * K, C)
    return sel_logits, sel_ids, sel_feats
